# baseline (speedup 1.0000x reference)
import jax
import jax.numpy as jnp
from jax.experimental import pallas as pl
from jax.experimental.pallas import tpu as pltpu

_BM = 1000


def _fused_heads(x_ref, wc_ref, bc_ref, wb_ref, bb_ref, sc_ref, pd_ref):
    x = x_ref[...]
    sc_ref[...] = (
        jnp.dot(x, wc_ref[...], preferred_element_type=jnp.float32) + bc_ref[...]
    )
    pd_ref[...] = (
        jnp.dot(x, wb_ref[...], preferred_element_type=jnp.float32) + bb_ref[...]
    )


def kernel(x, Wc, bc, Wb, bb):
    if x.ndim > 2:
        x = x.reshape(x.shape[0], -1)
    n, k = x.shape
    nc = Wc.shape[0]
    nb = Wb.shape[0]
    ncp = ((nc + 127) // 128) * 128
    nbp = ((nb + 127) // 128) * 128
    wc_t = jnp.zeros((k, ncp), x.dtype).at[:, :nc].set(Wc.T)
    wb_t = jnp.zeros((k, nbp), x.dtype).at[:, :nb].set(Wb.T)
    bc_p = jnp.zeros((1, ncp), x.dtype).at[0, :nc].set(bc)
    bb_p = jnp.zeros((1, nbp), x.dtype).at[0, :nb].set(bb)
    scores_p, deltas_p = pl.pallas_call(
        _fused_heads,
        grid=(pl.cdiv(n, _BM),),
        in_specs=[
            pl.BlockSpec((_BM, k), lambda i: (i, 0)),
            pl.BlockSpec((k, ncp), lambda i: (0, 0)),
            pl.BlockSpec((1, ncp), lambda i: (0, 0)),
            pl.BlockSpec((k, nbp), lambda i: (0, 0)),
            pl.BlockSpec((1, nbp), lambda i: (0, 0)),
        ],
        out_specs=[
            pl.BlockSpec((_BM, ncp), lambda i: (i, 0)),
            pl.BlockSpec((_BM, nbp), lambda i: (i, 0)),
        ],
        out_shape=[
            jax.ShapeDtypeStruct((n, ncp), x.dtype),
            jax.ShapeDtypeStruct((n, nbp), x.dtype),
        ],
        compiler_params=pltpu.CompilerParams(
            dimension_semantics=("parallel",),
        ),
    )(x, wc_t, bc_p, wb_t, bb_p)
    return (scores_p[:, :nc], deltas_p[:, :nb])


# speedup vs baseline: 4.4030x; 4.4030x over previous
import jax
import jax.numpy as jnp
from jax.experimental import pallas as pl
from jax.experimental.pallas import tpu as pltpu

_BM = 2048


def _fused_heads_t(x_ref, wc_ref, bc_ref, wb_ref, bb_ref, sc_ref, pd_ref):
    xt = x_ref[...]
    dims = (((1,), (1,)), ((), ()))
    sc_ref[...] = (
        jax.lax.dot_general(wc_ref[...], xt, dims, preferred_element_type=jnp.float32)
        + bc_ref[...]
    )
    pd_ref[...] = (
        jax.lax.dot_general(wb_ref[...], xt, dims, preferred_element_type=jnp.float32)
        + bb_ref[...]
    )


def kernel(x, Wc, bc, Wb, bb):
    if x.ndim > 2:
        x = x.reshape(x.shape[0], -1)
    n, k = x.shape
    nc = Wc.shape[0]
    nb = Wb.shape[0]
    scores_t, deltas_t = pl.pallas_call(
        _fused_heads_t,
        grid=(pl.cdiv(n, _BM),),
        in_specs=[
            pl.BlockSpec((_BM, k), lambda i: (i, 0)),
            pl.BlockSpec((nc, k), lambda i: (0, 0)),
            pl.BlockSpec((nc, 1), lambda i: (0, 0)),
            pl.BlockSpec((nb, k), lambda i: (0, 0)),
            pl.BlockSpec((nb, 1), lambda i: (0, 0)),
        ],
        out_specs=[
            pl.BlockSpec((nc, _BM), lambda i: (0, i)),
            pl.BlockSpec((nb, _BM), lambda i: (0, i)),
        ],
        out_shape=[
            jax.ShapeDtypeStruct((nc, n), x.dtype),
            jax.ShapeDtypeStruct((nb, n), x.dtype),
        ],
        compiler_params=pltpu.CompilerParams(
            dimension_semantics=("parallel",),
        ),
    )(x, Wc, bc.reshape(nc, 1), Wb, bb.reshape(nb, 1))
    return (scores_t.T, deltas_t.T)
